# Initial kernel scaffold; baseline (speedup 1.0000x reference)
import jax
import jax.numpy as jnp
from jax.experimental import pallas as pl


def kernel(x, edge_index, edge_weight, W, b):
    raise NotImplementedError("write your pallas kernel here")


# speedup vs baseline: 4.3586x; 4.3586x over previous
import functools

import jax
import jax.numpy as jnp
from jax import lax
from jax.experimental import pallas as pl
from jax.experimental.pallas import tpu as pltpu
from jax.experimental.pallas import tpu_sc as plsc

_NC = 2
_NS = 16
_NW = _NC * _NS
_CH = 128
_L = 16


def _linear(x, W, b):
    n, d_in = x.shape
    d_out = W.shape[0]
    blk = 2000

    def body(x_ref, wt_ref, b_ref, h_ref):
        h_ref[...] = (
            jnp.dot(x_ref[...], wt_ref[...], preferred_element_type=jnp.float32)
            + b_ref[...]
        )

    return pl.pallas_call(
        body,
        grid=(n // blk,),
        in_specs=[
            pl.BlockSpec((blk, d_in), lambda i: (i, 0)),
            pl.BlockSpec((d_in, d_out), lambda i: (0, 0)),
            pl.BlockSpec((1, d_out), lambda i: (0, 0)),
        ],
        out_specs=pl.BlockSpec((blk, d_out), lambda i: (i, 0)),
        out_shape=jax.ShapeDtypeStruct((n, d_out), jnp.float32),
    )(x, W.T, b[None, :])


def _combine(partials):
    nc, n, d = partials.shape
    blk = 2000

    def body(p_ref, o_ref):
        o_ref[...] = p_ref[0] + p_ref[1]

    return pl.pallas_call(
        body,
        grid=(n // blk,),
        in_specs=[pl.BlockSpec((nc, blk, d), lambda i: (0, i, 0))],
        out_specs=pl.BlockSpec((blk, d), lambda i: (i, 0)),
        out_shape=jax.ShapeDtypeStruct((n, d), jnp.float32),
    )(partials)


def _sc_aggregate(h, src, dst, w, zeros):
    n, d = h.shape
    e = src.shape[0]
    assert e % _CH == 0
    n_chunks = e // _CH
    base_trips = n_chunks // _NW
    extra = n_chunks % _NW
    rows_per_tile = (n // _NS) // 8 * 8
    tail_rows = n - _NS * rows_per_tile
    assert tail_rows % 8 == 0
    mesh = plsc.VectorSubcoreMesh(core_axis_name="c", subcore_axis_name="s")

    @functools.partial(
        pl.kernel,
        out_type=jax.ShapeDtypeStruct((_NC, n, d), jnp.float32),
        mesh=mesh,
        compiler_params=pltpu.CompilerParams(needs_layout_passes=False),
        scratch_types=[
            pltpu.VMEM((_CH,), jnp.int32),
            pltpu.VMEM((_CH,), jnp.int32),
            pltpu.VMEM((_CH,), jnp.float32),
            pltpu.VMEM((_CH, d), jnp.float32),
            pltpu.VMEM_SHARED((n, d), jnp.float32),
            pltpu.SemaphoreType.DMA,
        ],
    )
    def agg(h_hbm, src_hbm, dst_hbm, w_hbm, z_hbm, out_hbm,
            src_v, dst_v, w_v, rows_v, accum, sem):
        cid = lax.axis_index("c")
        sid = lax.axis_index("s")
        wid = cid * _NS + sid

        r0 = sid * rows_per_tile
        pltpu.sync_copy(z_hbm.at[pl.ds(r0, rows_per_tile)],
                        accum.at[pl.ds(r0, rows_per_tile)])
        if tail_rows:
            @pl.when(sid == _NS - 1)
            def _zero_tail():
                t0 = _NS * rows_per_tile
                pltpu.sync_copy(z_hbm.at[pl.ds(t0, tail_rows)],
                                accum.at[pl.ds(t0, tail_rows)])
        plsc.subcore_barrier()

        ntrips = base_trips + jnp.where(wid < extra, 1, 0)

        def body(j, carry):
            c = wid + _NW * j
            e0 = c * _CH
            pltpu.sync_copy(src_hbm.at[pl.ds(e0, _CH)], src_v)
            pltpu.sync_copy(dst_hbm.at[pl.ds(e0, _CH)], dst_v)
            pltpu.sync_copy(w_hbm.at[pl.ds(e0, _CH)], w_v)
            pltpu.async_copy(h_hbm.at[src_v], rows_v, sem).wait()

            def scale(i, c2):
                wv = plsc.load_gather(w_v, [jnp.full((_L,), i, jnp.int32)])
                for f in range(d // _L):
                    sl = pl.ds(f * _L, _L)
                    rows_v[i, sl] = rows_v[i, sl] * wv
                return c2

            lax.fori_loop(0, _CH, scale, 0)
            pltpu.sync_copy(rows_v, accum.at[dst_v], add=True)
            return carry

        lax.fori_loop(0, ntrips, body, 0)

        plsc.subcore_barrier()
        pltpu.sync_copy(accum.at[pl.ds(r0, rows_per_tile)],
                        out_hbm.at[cid, pl.ds(r0, rows_per_tile)])
        if tail_rows:
            @pl.when(sid == _NS - 1)
            def _write_tail():
                t0 = _NS * rows_per_tile
                pltpu.sync_copy(accum.at[pl.ds(t0, tail_rows)],
                                out_hbm.at[cid, pl.ds(t0, tail_rows)])

    return agg(h, src, dst, w, zeros)


def kernel(x, edge_index, edge_weight, W, b):
    h = _linear(x, W, b)
    zeros = jnp.zeros_like(h)
    partials = _sc_aggregate(h, edge_index[0], edge_index[1], edge_weight, zeros)
    return _combine(partials)
